# baseline (speedup 1.0000x reference)
import functools

import jax
import jax.numpy as jnp
from jax.experimental import pallas as pl
from jax.experimental.pallas import tpu as pltpu

_BF = jnp.bfloat16
_F32 = jnp.float32


def _whole(arr):
    zeros = (0,) * arr.ndim
    return pl.BlockSpec(arr.shape, lambda b: zeros)


def _bf(ref):
    return ref[...].astype(_BF)


def _tnet_kernel(xt_ref, w1_ref, w2_ref, b2_ref, w3_ref, b3_ref, o_ref,
                 *, nb):
    N = xt_ref.shape[1]
    x = xt_ref[...].reshape(nb * N, 8)
    h = jnp.maximum(jnp.dot(x.astype(_BF), _bf(w1_ref),
                            preferred_element_type=_F32), 0.0)
    h = jnp.maximum(jnp.dot(h.astype(_BF), _bf(w2_ref),
                            preferred_element_type=_F32) + b2_ref[...], 0.0)
    h = jnp.maximum(jnp.dot(h.astype(_BF), _bf(w3_ref),
                            preferred_element_type=_F32) + b3_ref[...], 0.0)
    for b in range(nb):
        o_ref[b] = jnp.max(h[b * N:(b + 1) * N], axis=0, keepdims=True)


def _knn_edge_kernel(xt_ref, xx_ref, p1_ref, p2_ref, wl_ref, wq_ref, bp_ref,
                     wn_ref, w2_ref, b2_ref, w3_ref, b3_ref, w4_ref, b4_ref,
                     w5_ref, b5_ref, w6_ref, b6_ref, o_ref, s_ref, *, k, nb):
    N = xt_ref.shape[1]
    G = N // 128
    NEG = jnp.float32(-3e38)
    row = jax.lax.broadcasted_iota(jnp.int32, (N, N), 0)
    col = jax.lax.broadcasted_iota(jnp.int32, (N, N), 1)
    qms = []
    pres = []
    for b in range(nb):
        xt = xt_ref[b]
        xtb = xt.astype(_BF)
        inner = jax.lax.dot_general(xtb, xtb, (((1,), (1,)), ((), ())),
                                    preferred_element_type=_F32)
        s = jnp.where(row == col, NEG, 2.0 * inner - xx_ref[b])
        stack = []
        for g in range(G):
            bits = jax.lax.bitcast_convert_type(s[:, 128 * g:128 * (g + 1)],
                                                jnp.int32)
            stack.append(jax.lax.bitcast_convert_type(
                (bits & ~7) | g, _F32))
        for r in range(G):
            pairs = [(i, i + 1) for i in range(r % 2, G - 1, 2)]
            for a, c in pairs:
                hi = jnp.maximum(stack[a], stack[c])
                lo = jnp.minimum(stack[a], stack[c])
                stack[a], stack[c] = hi, lo
        for g in range(G):
            s_ref[b, g] = stack[g]
        q_self = jnp.dot(xtb, _bf(wn_ref), preferred_element_type=_F32)
        qms.append(q_self)
        pres.append(q_self)

    def _step(qmaxs, write_mask):
        out = []
        for b in range(nb):
            cur = s_ref[b, 0]
            m = jnp.max(cur, axis=1, keepdims=True)
            pred = cur == m
            ohl = jnp.where(pred, 1.0, 0.0)
            gid = jax.lax.bitcast_convert_type(m, jnp.int32) & 7
            qt = None
            for g in range(G):
                c = jnp.dot(ohl, pres[b][128 * g:128 * (g + 1)],
                            preferred_element_type=_F32)
                c = jnp.where(gid == g, c, 0.0)
                qt = c if qt is None else qt + c
            if write_mask:
                for g in range(G - 1):
                    s_ref[b, g] = jnp.where(pred, s_ref[b, g + 1],
                                            s_ref[b, g])
                s_ref[b, G - 1] = jnp.where(pred, NEG, s_ref[b, G - 1])
            out.append(jnp.maximum(qmaxs[b], qt))
        return tuple(out)

    qms = jax.lax.fori_loop(0, k - 2, lambda _, q: _step(q, True),
                            tuple(qms))
    qms = _step(qms, False)

    hs = []
    for b in range(nb):
        xtb = xt_ref[b].astype(_BF)
        m6 = (jnp.dot(xtb, _bf(p1_ref), preferred_element_type=_F32)
              * jnp.dot(xtb, _bf(p2_ref), preferred_element_type=_F32))
        p = (jnp.dot(xtb, _bf(wl_ref), preferred_element_type=_F32)
             + jnp.dot(m6.astype(_BF), _bf(wq_ref),
                       preferred_element_type=_F32) + bp_ref[...])
        hs.append(jnp.maximum(p + qms[b], 0.0))
    h = jnp.concatenate(hs, axis=0)
    h = jnp.maximum(jnp.dot(h.astype(_BF), _bf(w2_ref),
                            preferred_element_type=_F32) + b2_ref[...], 0.0)
    h = jnp.maximum(jnp.dot(h.astype(_BF), _bf(w3_ref),
                            preferred_element_type=_F32) + b3_ref[...], 0.0)
    h = jnp.maximum(jnp.dot(h.astype(_BF), _bf(w4_ref),
                            preferred_element_type=_F32) + b4_ref[...], 0.0)
    h = jnp.maximum(jnp.dot(h.astype(_BF), _bf(w5_ref),
                            preferred_element_type=_F32) + b5_ref[...], 0.0)
    w6 = _bf(w6_ref)
    for b in range(nb):
        h6 = jnp.dot(h[b * N:(b + 1) * N].astype(_BF), w6,
                     preferred_element_type=_F32) + b6_ref[...]
        o_ref[b] = jnp.max(h6, axis=0, keepdims=True)


def _fc3_kernel(x_ref, w1_ref, b1_ref, w2_ref, b2_ref, w3_ref, b3_ref, o_ref,
                *, final):
    h = jnp.maximum(jnp.dot(x_ref[...].astype(_BF), _bf(w1_ref),
                            preferred_element_type=_F32) + b1_ref[...], 0.0)
    h = jnp.maximum(jnp.dot(h.astype(_BF), _bf(w2_ref),
                            preferred_element_type=_F32) + b2_ref[...], 0.0)
    z = jnp.dot(h.astype(_BF), _bf(w3_ref),
                preferred_element_type=_F32) + b3_ref[...]
    if final == "logsoftmax":
        z = z - jnp.max(z, axis=-1, keepdims=True)
        z = z - jnp.log(jnp.sum(jnp.exp(z), axis=-1, keepdims=True))
    o_ref[...] = z


def _tnet_conv_pool(x, t_w1, t_w2, t_b2, t_w3, t_b3, nb=8):
    B, _, N = x.shape
    xt = jnp.transpose(x, (0, 2, 1))
    xt = jnp.concatenate(
        [xt, jnp.ones((B, N, 1), _F32), jnp.zeros((B, N, 4), _F32)], axis=-1)
    consts = (t_w1, t_w2, t_b2, t_w3, t_b3)
    if B % nb:
        nb = 1
    pooled = pl.pallas_call(
        functools.partial(_tnet_kernel, nb=nb),
        out_shape=jax.ShapeDtypeStruct((B, 1, 1024), _F32),
        grid=(B // nb,),
        in_specs=[pl.BlockSpec((nb, N, 8), lambda b: (b, 0, 0))]
                 + [_whole(a) for a in consts],
        out_specs=pl.BlockSpec((nb, 1, 1024), lambda b: (b, 0, 0)),
        compiler_params=pltpu.CompilerParams(
            dimension_semantics=("parallel",)),
    )(xt, *consts)
    return pooled[:, 0, :]


def _fc3(x, layers, final="none", bm=128):
    (w1, b1), (w2, b2), (w3, b3) = layers
    M, K = x.shape
    if M % bm:
        bm = M
    Nout = w3.shape[1]
    consts = (w1, b1, w2, b2, w3, b3)
    return pl.pallas_call(
        functools.partial(_fc3_kernel, final=final),
        out_shape=jax.ShapeDtypeStruct((M, Nout), _F32),
        grid=(M // bm,),
        in_specs=[pl.BlockSpec((bm, K), lambda b: (b, 0))]
                 + [_whole(a) for a in consts],
        out_specs=pl.BlockSpec((bm, Nout), lambda b: (b, 0)),
        compiler_params=pltpu.CompilerParams(
            dimension_semantics=("parallel",)),
    )(x, *consts)


def _knn_edge_pool(params, xbt, k, nb=8):
    (m_wp, m_wn, m_w2, m_b2, m_w3, m_b3, m_w4, m_b4,
     m_w5, m_b5, m_w6, m_b6) = params
    B, N, _ = xbt.shape
    if B % nb:
        nb = 1
    xt = jnp.pad(xbt, ((0, 0), (0, 0), (0, 5)))
    xx = jnp.sum(xbt * xbt, axis=2)[:, None, :]
    sel_a = jnp.array([0, 1, 2, 0, 0, 1])
    sel_b = jnp.array([0, 1, 2, 1, 2, 2])
    eye8 = jnp.eye(8, dtype=_F32)
    p1 = eye8[:, sel_a]
    p2 = eye8[:, sel_b]
    wl = jnp.concatenate([m_wp[0:3] + m_wp[9:12],
                          jnp.zeros((5, 64), _F32)], axis=0)
    wq = m_wp[3:9]
    bp = m_wp[12:13]
    consts = (p1, p2, wl, wq, bp, m_wn, m_w2, m_b2, m_w3, m_b3,
              m_w4, m_b4, m_w5, m_b5, m_w6, m_b6)
    out = pl.pallas_call(
        functools.partial(_knn_edge_kernel, k=k, nb=nb),
        out_shape=jax.ShapeDtypeStruct((B, 1, 1024), _F32),
        grid=(B // nb,),
        in_specs=[pl.BlockSpec((nb, N, 8), lambda b: (b, 0, 0)),
                  pl.BlockSpec((nb, 1, N), lambda b: (b, 0, 0))]
                 + [_whole(a) for a in consts],
        out_specs=pl.BlockSpec((nb, 1, 1024), lambda b: (b, 0, 0)),
        scratch_shapes=[pltpu.VMEM((nb, N // 128, N, 128), _F32)],
        compiler_params=pltpu.CompilerParams(
            dimension_semantics=("parallel",)),
    )(xt, xx, *consts)
    return out[:, 0, :]


def kernel(x, t_w1, t_w2, t_b2, t_w3, t_b3,
           tfc_w1, tfc_b1, tfc_w2, tfc_b2, tfc_w3, tfc_b3,
           m_wp, m_wn, m_w2, m_b2, m_w3, m_b3, m_w4, m_b4, m_w5, m_b5,
           m_w6, m_b6,
           hfc_w1, hfc_b1, hfc_w2, hfc_b2, hfc_w3, hfc_b3):
    B, _, N = x.shape
    k = 20

    pooled = _tnet_conv_pool(x, t_w1, t_w2, t_b2, t_w3, t_b3)
    tout = _fc3(pooled, ((tfc_w1, tfc_b1), (tfc_w2, tfc_b2),
                         (tfc_w3, tfc_b3)), final="none")
    m3 = tout.reshape(B, 3, 3) + jnp.eye(3, dtype=_F32)[None]
    xbt = jnp.einsum("bcn,bcd->bnd", x, m3)

    edge_params = (m_wp, m_wn, m_w2, m_b2, m_w3, m_b3, m_w4, m_b4,
                   m_w5, m_b5, m_w6, m_b6)
    g = _knn_edge_pool(edge_params, xbt, k)
    return _fc3(g, ((hfc_w1, hfc_b1), (hfc_w2, hfc_b2),
                    (hfc_w3, hfc_b3)), final="logsoftmax")


# speedup vs baseline: 1.1865x; 1.1865x over previous
import functools

import jax
import jax.numpy as jnp
from jax.experimental import pallas as pl
from jax.experimental.pallas import tpu as pltpu

_BF = jnp.bfloat16
_F32 = jnp.float32


def _whole(arr):
    zeros = (0,) * arr.ndim
    return pl.BlockSpec(arr.shape, lambda b: zeros)


def _bf(ref):
    return ref[...].astype(_BF)


def _tnet_kernel(xt_ref, w1_ref, w2_ref, b2_ref, w3_ref, b3_ref, o_ref,
                 *, nb):
    N = xt_ref.shape[1]
    x = xt_ref[...].reshape(nb * N, 8)
    h = jnp.maximum(jnp.dot(x.astype(_BF), _bf(w1_ref),
                            preferred_element_type=_F32), 0.0)
    h = jnp.maximum(jnp.dot(h.astype(_BF), _bf(w2_ref),
                            preferred_element_type=_F32) + b2_ref[...], 0.0)
    h = jnp.maximum(jnp.dot(h.astype(_BF), _bf(w3_ref),
                            preferred_element_type=_F32) + b3_ref[...], 0.0)
    for b in range(nb):
        o_ref[b] = jnp.max(h[b * N:(b + 1) * N], axis=0, keepdims=True)


def _knn_edge_kernel(xt_ref, xx_ref, p1_ref, p2_ref, wl_ref, wq_ref, bp_ref,
                     wn_ref, w2_ref, b2_ref, w3_ref, b3_ref, w4_ref, b4_ref,
                     w5_ref, b5_ref, w6_ref, b6_ref, o_ref, s_ref, *, k, nb):
    N = xt_ref.shape[1]
    row = jax.lax.broadcasted_iota(jnp.int32, (N, N), 0)
    col = jax.lax.broadcasted_iota(jnp.int32, (N, N), 1)
    qms = []
    pres = []
    for b in range(nb):
        xt = xt_ref[b]
        xtb = xt.astype(_BF)
        inner = jax.lax.dot_general(xtb, xtb, (((1,), (1,)), ((), ())),
                                    preferred_element_type=_F32)
        s_ref[b] = jnp.where(row == col, -jnp.inf, 2.0 * inner - xx_ref[b])
        q_self = jnp.dot(xtb, _bf(wn_ref), preferred_element_type=_F32)
        qms.append(q_self)
        pres.append(q_self)

    def _step(qmaxs, write_mask):
        out = []
        for b in range(nb):
            s = s_ref[b]
            m = jnp.max(s, axis=1, keepdims=True)
            pred = s == m
            qt = jnp.dot(jnp.where(pred, 1.0, 0.0), pres[b],
                         preferred_element_type=_F32)
            if write_mask:
                s_ref[b] = jnp.where(pred, -jnp.inf, s)
            out.append(jnp.maximum(qmaxs[b], qt))
        return tuple(out)

    qms = jax.lax.fori_loop(0, k - 2, lambda _, q: _step(q, True),
                            tuple(qms))
    qms = _step(qms, False)

    hs = []
    for b in range(nb):
        xtb = xt_ref[b].astype(_BF)
        m6 = (jnp.dot(xtb, _bf(p1_ref), preferred_element_type=_F32)
              * jnp.dot(xtb, _bf(p2_ref), preferred_element_type=_F32))
        p = (jnp.dot(xtb, _bf(wl_ref), preferred_element_type=_F32)
             + jnp.dot(m6.astype(_BF), _bf(wq_ref),
                       preferred_element_type=_F32) + bp_ref[...])
        hs.append(jnp.maximum(p + qms[b], 0.0))
    h = jnp.concatenate(hs, axis=0)
    h = jnp.maximum(jnp.dot(h.astype(_BF), _bf(w2_ref),
                            preferred_element_type=_F32) + b2_ref[...], 0.0)
    h = jnp.maximum(jnp.dot(h.astype(_BF), _bf(w3_ref),
                            preferred_element_type=_F32) + b3_ref[...], 0.0)
    h = jnp.maximum(jnp.dot(h.astype(_BF), _bf(w4_ref),
                            preferred_element_type=_F32) + b4_ref[...], 0.0)
    h = jnp.maximum(jnp.dot(h.astype(_BF), _bf(w5_ref),
                            preferred_element_type=_F32) + b5_ref[...], 0.0)
    w6 = _bf(w6_ref)
    for b in range(nb):
        h6 = jnp.dot(h[b * N:(b + 1) * N].astype(_BF), w6,
                     preferred_element_type=_F32) + b6_ref[...]
        o_ref[b] = jnp.max(h6, axis=0, keepdims=True)


def _fc3_kernel(x_ref, w1_ref, b1_ref, w2_ref, b2_ref, w3_ref, b3_ref, o_ref,
                *, final):
    h = jnp.maximum(jnp.dot(x_ref[...].astype(_BF), _bf(w1_ref),
                            preferred_element_type=_F32) + b1_ref[...], 0.0)
    h = jnp.maximum(jnp.dot(h.astype(_BF), _bf(w2_ref),
                            preferred_element_type=_F32) + b2_ref[...], 0.0)
    z = jnp.dot(h.astype(_BF), _bf(w3_ref),
                preferred_element_type=_F32) + b3_ref[...]
    if final == "logsoftmax":
        z = z - jnp.max(z, axis=-1, keepdims=True)
        z = z - jnp.log(jnp.sum(jnp.exp(z), axis=-1, keepdims=True))
    o_ref[...] = z


def _tnet_conv_pool(x, t_w1, t_w2, t_b2, t_w3, t_b3, nb=8):
    B, _, N = x.shape
    xt = jnp.transpose(x, (0, 2, 1))
    xt = jnp.concatenate(
        [xt, jnp.ones((B, N, 1), _F32), jnp.zeros((B, N, 4), _F32)], axis=-1)
    consts = (t_w1, t_w2, t_b2, t_w3, t_b3)
    if B % nb:
        nb = 1
    pooled = pl.pallas_call(
        functools.partial(_tnet_kernel, nb=nb),
        out_shape=jax.ShapeDtypeStruct((B, 1, 1024), _F32),
        grid=(B // nb,),
        in_specs=[pl.BlockSpec((nb, N, 8), lambda b: (b, 0, 0))]
                 + [_whole(a) for a in consts],
        out_specs=pl.BlockSpec((nb, 1, 1024), lambda b: (b, 0, 0)),
        compiler_params=pltpu.CompilerParams(
            dimension_semantics=("parallel",)),
    )(xt, *consts)
    return pooled[:, 0, :]


def _fc3(x, layers, final="none", bm=128):
    (w1, b1), (w2, b2), (w3, b3) = layers
    M, K = x.shape
    if M % bm:
        bm = M
    Nout = w3.shape[1]
    consts = (w1, b1, w2, b2, w3, b3)
    return pl.pallas_call(
        functools.partial(_fc3_kernel, final=final),
        out_shape=jax.ShapeDtypeStruct((M, Nout), _F32),
        grid=(M // bm,),
        in_specs=[pl.BlockSpec((bm, K), lambda b: (b, 0))]
                 + [_whole(a) for a in consts],
        out_specs=pl.BlockSpec((bm, Nout), lambda b: (b, 0)),
        compiler_params=pltpu.CompilerParams(
            dimension_semantics=("parallel",)),
    )(x, *consts)


def _knn_edge_pool(params, xbt, k, nb=8):
    (m_wp, m_wn, m_w2, m_b2, m_w3, m_b3, m_w4, m_b4,
     m_w5, m_b5, m_w6, m_b6) = params
    B, N, _ = xbt.shape
    if B % nb:
        nb = 1
    xt = jnp.pad(xbt, ((0, 0), (0, 0), (0, 5)))
    xx = jnp.sum(xbt * xbt, axis=2)[:, None, :]
    sel_a = jnp.array([0, 1, 2, 0, 0, 1])
    sel_b = jnp.array([0, 1, 2, 1, 2, 2])
    eye8 = jnp.eye(8, dtype=_F32)
    p1 = eye8[:, sel_a]
    p2 = eye8[:, sel_b]
    wl = jnp.concatenate([m_wp[0:3] + m_wp[9:12],
                          jnp.zeros((5, 64), _F32)], axis=0)
    wq = m_wp[3:9]
    bp = m_wp[12:13]
    consts = (p1, p2, wl, wq, bp, m_wn, m_w2, m_b2, m_w3, m_b3,
              m_w4, m_b4, m_w5, m_b5, m_w6, m_b6)
    out = pl.pallas_call(
        functools.partial(_knn_edge_kernel, k=k, nb=nb),
        out_shape=jax.ShapeDtypeStruct((B, 1, 1024), _F32),
        grid=(B // nb,),
        in_specs=[pl.BlockSpec((nb, N, 8), lambda b: (b, 0, 0)),
                  pl.BlockSpec((nb, 1, N), lambda b: (b, 0, 0))]
                 + [_whole(a) for a in consts],
        out_specs=pl.BlockSpec((nb, 1, 1024), lambda b: (b, 0, 0)),
        scratch_shapes=[pltpu.VMEM((nb, N, N), _F32)],
        compiler_params=pltpu.CompilerParams(
            dimension_semantics=("parallel",)),
    )(xt, xx, *consts)
    return out[:, 0, :]


def kernel(x, t_w1, t_w2, t_b2, t_w3, t_b3,
           tfc_w1, tfc_b1, tfc_w2, tfc_b2, tfc_w3, tfc_b3,
           m_wp, m_wn, m_w2, m_b2, m_w3, m_b3, m_w4, m_b4, m_w5, m_b5,
           m_w6, m_b6,
           hfc_w1, hfc_b1, hfc_w2, hfc_b2, hfc_w3, hfc_b3):
    B, _, N = x.shape
    k = 20

    pooled = _tnet_conv_pool(x, t_w1, t_w2, t_b2, t_w3, t_b3)
    tout = _fc3(pooled, ((tfc_w1, tfc_b1), (tfc_w2, tfc_b2),
                         (tfc_w3, tfc_b3)), final="none")
    m3 = tout.reshape(B, 3, 3) + jnp.eye(3, dtype=_F32)[None]
    xbt = jnp.einsum("bcn,bcd->bnd", x, m3)

    edge_params = (m_wp, m_wn, m_w2, m_b2, m_w3, m_b3, m_w4, m_b4,
                   m_w5, m_b5, m_w6, m_b6)
    g = _knn_edge_pool(edge_params, xbt, k)
    return _fc3(g, ((hfc_w1, hfc_b1), (hfc_w2, hfc_b2),
                    (hfc_w3, hfc_b3)), final="logsoftmax")
